# baseline (speedup 1.0000x reference)
import functools

import jax
import jax.numpy as jnp
from jax import lax
from jax.experimental import pallas as pl
from jax.experimental.pallas import tpu as pltpu
from jax.experimental.pallas import tpu_sc as plsc

N = 50000
E = 800000
T = 4
IN_F = 11
H = 128
OUT_F = 3

NPAD = 50176
QROWS = 12544
BQ = 80
NBQ = 192
CAPB = 16 * NBQ * BQ
EPB = 4 * CAPB
NB_X = EPB // (32 * 320)
BATCH = 320
ROWS_PER_TILE = NPAD // 16
R = 512
GRID = NPAD // R

f32 = jnp.float32


WCH = 112
NCH = ROWS_PER_TILE // WCH
CH = 8


def _zero_acc(acc, zeros, wbuf, row0):
  pltpu.sync_copy(zeros.at[pl.ds(0, WCH)], wbuf)
  for i in range(NCH):
    pltpu.sync_copy(wbuf, acc.at[pl.ds(row0 + i * WCH, WCH)])


def _writeout(acc, out, wbuf, row0):
  for i in range(NCH):
    pltpu.sync_copy(acc.at[pl.ds(row0 + i * WCH, WCH)], wbuf)
    pltpu.sync_copy(wbuf, out.at[pl.ds(row0 + i * WCH, WCH)])


def _pipeline_chunk(tbl, acc, idxs, idxd, bufs, gsems, ssems):
  gd = [pltpu.async_copy(tbl.at[idxs.at[k]], bufs[k & 1], gsems[k & 1])
        for k in range(2)]
  sd = {}
  for k in range(CH):
    buf = bufs[k & 1]
    gd[k].wait()
    sd[k] = pltpu.async_copy(buf, acc.at[idxd.at[k]], ssems[k & 1], add=True)
    if k + 2 < CH:
      sd[k].wait()
      gd.append(pltpu.async_copy(tbl.at[idxs.at[k + 2]], buf, gsems[k & 1]))
  sd[CH - 2].wait()
  sd[CH - 1].wait()


def _deg_body(srcx, zeros16, ones_h, outa, outb, idx, ones_v, wbuf, sem, acc):
  c = lax.axis_index("c")
  s = lax.axis_index("s")
  w = c * 16 + s
  row0 = s * ROWS_PER_TILE
  _zero_acc(acc, zeros16, wbuf, row0)
  pltpu.sync_copy(ones_h, ones_v)
  plsc.subcore_barrier()

  def chunk(j, _):
    pltpu.sync_copy(srcx.at[w, pl.ds(j * CH, CH)], idx)
    descs = [
        pltpu.async_copy(ones_v, acc.at[idx.at[k]], sem, add=True)
        for k in range(CH)
    ]
    for d in descs:
      d.wait()
    return 0

  lax.fori_loop(0, NB_X // CH, chunk, 0)
  plsc.subcore_barrier()

  @pl.when(c == 0)
  def _():
    _writeout(acc, outa, wbuf, row0)

  @pl.when(c == 1)
  def _():
    _writeout(acc, outb, wbuf, row0)


def _seg16_body(g, srcx, dstx, zeros16, outa, outb, idxs, idxd, b0, b1, wbuf,
                gsem0, gsem1, ssem0, ssem1, acc):
  c = lax.axis_index("c")
  s = lax.axis_index("s")
  w = c * 16 + s
  row0 = s * ROWS_PER_TILE
  _zero_acc(acc, zeros16, wbuf, row0)
  plsc.subcore_barrier()

  def chunk(j, _):
    pltpu.sync_copy(srcx.at[w, pl.ds(j * CH, CH)], idxs)
    pltpu.sync_copy(dstx.at[w, pl.ds(j * CH, CH)], idxd)
    _pipeline_chunk(g, acc, idxs, idxd, (b0, b1), (gsem0, gsem1),
                    (ssem0, ssem1))
    return 0

  lax.fori_loop(0, NB_X // CH, chunk, 0)
  plsc.subcore_barrier()

  @pl.when(c == 0)
  def _():
    _writeout(acc, outa, wbuf, row0)

  @pl.when(c == 1)
  def _():
    _writeout(acc, outb, wbuf, row0)


WQ = 28
NQCH = QROWS // 16 // WQ


def _seg128_body(g, srcq, dstq, zeros_w, out, idxs, idxd, b0, b1, wbuf,
                 gsem0, gsem1, ssem0, ssem1, acc):
  c = lax.axis_index("c")
  s = lax.axis_index("s")
  row0 = s * (QROWS // 16)

  def edge_scan(blk):
    def chunk(j, _):
      pltpu.sync_copy(srcq.at[blk, s, pl.ds(j * CH, CH)], idxs)
      pltpu.sync_copy(dstq.at[blk, s, pl.ds(j * CH, CH)], idxd)
      _pipeline_chunk(g, acc, idxs, idxd, (b0, b1), (gsem0, gsem1),
                      (ssem0, ssem1))
      return 0

    lax.fori_loop(0, NBQ // CH, chunk, 0)

  for p in range(2):
    pltpu.sync_copy(zeros_w, wbuf)
    for i in range(NQCH):
      pltpu.sync_copy(wbuf, acc.at[pl.ds(row0 + i * WQ, WQ)])
    plsc.subcore_barrier()
    for cc in range(2):
      @pl.when(c == cc)
      def _(blk=2 * p + cc):
        edge_scan(blk)
    plsc.subcore_barrier()
    for cc in range(2):
      @pl.when(c == cc)
      def _(blk=2 * p + cc):
        base = blk * QROWS + row0
        for i in range(NQCH):
          pltpu.sync_copy(acc.at[pl.ds(row0 + i * WQ, WQ)], wbuf)
          pltpu.sync_copy(wbuf, out.at[pl.ds(base + i * WQ, WQ)])


def _sc_mesh():
  return plsc.VectorSubcoreMesh(core_axis_name="c", subcore_axis_name="s")


_SC_PARAMS = pltpu.CompilerParams(use_tc_tiling_on_sc=False)


def _sc_deg(srcx, zeros16, ones_h):
  k = pl.kernel(
      _deg_body,
      out_type=(jax.ShapeDtypeStruct((NPAD, 16), f32),
                jax.ShapeDtypeStruct((NPAD, 16), f32)),
      mesh=_sc_mesh(),
      compiler_params=_SC_PARAMS,
      scratch_types=[
          pltpu.VMEM((CH, BATCH), jnp.int32),
          pltpu.VMEM((BATCH, 16), f32),
          pltpu.VMEM((WCH, 16), f32),
          pltpu.SemaphoreType.DMA,
          pltpu.VMEM_SHARED((NPAD, 16), f32),
      ],
  )
  return k(srcx, zeros16, ones_h)


def _sc_seg16(g, srcx, dstx, zeros16):
  k = pl.kernel(
      _seg16_body,
      out_type=(jax.ShapeDtypeStruct((NPAD, 16), f32),
                jax.ShapeDtypeStruct((NPAD, 16), f32)),
      mesh=_sc_mesh(),
      compiler_params=_SC_PARAMS,
      scratch_types=[
          pltpu.VMEM((CH, BATCH), jnp.int32),
          pltpu.VMEM((CH, BATCH), jnp.int32),
          pltpu.VMEM((BATCH, 16), f32),
          pltpu.VMEM((BATCH, 16), f32),
          pltpu.VMEM((WCH, 16), f32),
          pltpu.SemaphoreType.DMA,
          pltpu.SemaphoreType.DMA,
          pltpu.SemaphoreType.DMA,
          pltpu.SemaphoreType.DMA,
          pltpu.VMEM_SHARED((NPAD, 16), f32),
      ],
  )
  return k(g, srcx, dstx, zeros16)


def _sc_seg128(g, srcq, dstq, zeros_w):
  k = pl.kernel(
      _seg128_body,
      out_type=jax.ShapeDtypeStruct((NPAD, H), f32),
      mesh=_sc_mesh(),
      compiler_params=_SC_PARAMS,
      scratch_types=[
          pltpu.VMEM((CH, BQ), jnp.int32),
          pltpu.VMEM((CH, BQ), jnp.int32),
          pltpu.VMEM((BQ, H), f32),
          pltpu.VMEM((BQ, H), f32),
          pltpu.VMEM((WQ, H), f32),
          pltpu.SemaphoreType.DMA,
          pltpu.SemaphoreType.DMA,
          pltpu.SemaphoreType.DMA,
          pltpu.SemaphoreType.DMA,
          pltpu.VMEM_SHARED((QROWS, H), f32),
      ],
  )
  return k(g, srcq, dstq, zeros_w)


def _row_spec(k):
  return pl.BlockSpec((R, k), lambda i: (i, 0))


def _full_spec(a, b):
  return pl.BlockSpec((a, b), lambda i: (0, 0))


def _tc_call(body, in_specs, out_specs, out_shapes, args):
  return pl.pallas_call(
      body,
      grid=(GRID,),
      in_specs=in_specs,
      out_specs=out_specs,
      out_shape=out_shapes,
      compiler_params=pltpu.CompilerParams(
          dimension_semantics=("arbitrary",)),
  )(*args)


def _prep0_body(dega, degb, x0, dis_o, gx_o):
  deg = dega[...][:, 0:1] + degb[...][:, 0:1]
  d = jnp.where(deg > 0, lax.rsqrt(jnp.maximum(deg, 1.0)), 0.0)
  rowid = (lax.broadcasted_iota(jnp.int32, (R, 1), 0)
           + pl.program_id(0) * R)
  d = jnp.where(rowid < N, d, 0.0)
  dis_o[...] = d
  gx_o[...] = d * x0[...]


def _tc_prep0(dega, degb, x0):
  return _tc_call(
      _prep0_body,
      [_row_spec(16), _row_spec(16), _row_spec(16)],
      [_row_spec(1), _row_spec(16)],
      (jax.ShapeDtypeStruct((NPAD, 1), f32),
       jax.ShapeDtypeStruct((NPAD, 16), f32)),
      (dega, degb, x0),
  )


def _nextx(u, x, xn_raw, dis, c):
  dt8 = jnp.dot(xn_raw - x, c["e6r"], preferred_element_type=f32)
  v8 = (u - jnp.dot(x, c["q36"], preferred_element_type=f32)) / dt8
  xn = (xn_raw * c["cmask"]
        + jnp.dot(u, c["p1"], preferred_element_type=f32)
        + jnp.dot(v8, c["p2"], preferred_element_type=f32))
  return xn


def _step0_body(x0, sxa, sxb, dis, xn_raw, w0x3, w1x3, bz, bh, wh8, bh8, e6r,
                q36, p1, p2, cmask, h_o, u_o, x_o, gx_o, gh_o):
  x = x0[...]
  d = dis[...]
  a = -(d * (sxa[...] + sxb[...]))
  u1 = (jnp.dot(x, w0x3[...], preferred_element_type=f32)
        + jnp.dot(a, w1x3[...], preferred_element_type=f32))
  z = jax.nn.sigmoid(u1[:, 0:H] + bz[...])
  ht = jnp.tanh(u1[:, 2 * H:3 * H] + bh[...])
  h1 = (1.0 - z) * ht
  u = jnp.dot(h1, wh8[...], preferred_element_type=f32) + bh8[...]
  c = {"e6r": e6r[...], "q36": q36[...], "p1": p1[...], "p2": p2[...],
       "cmask": cmask[...]}
  xn = _nextx(u, x, xn_raw[...], d, c)
  real = (lax.broadcasted_iota(jnp.int32, (R, 1), 0)
          + pl.program_id(0) * R) < N
  h_o[...] = h1
  u_o[...] = u
  x_o[...] = xn
  gx_o[...] = jnp.where(real, d * xn, 0.0)
  gh_o[...] = jnp.where(real, d * h1, 0.0)


def _tc_step0(x0, sxa, sxb, dis, xn_raw, cw):
  return _tc_call(
      _step0_body,
      [_row_spec(16), _row_spec(16), _row_spec(16), _row_spec(1),
       _row_spec(16), _full_spec(16, 3 * H), _full_spec(16, 3 * H),
       _full_spec(1, H), _full_spec(1, H), _full_spec(H, 8), _full_spec(1, 8),
       _full_spec(16, 8), _full_spec(16, 8), _full_spec(8, 16),
       _full_spec(8, 16), _full_spec(1, 16)],
      [_row_spec(H), _row_spec(8), _row_spec(16), _row_spec(16),
       _row_spec(H)],
      (jax.ShapeDtypeStruct((NPAD, H), f32),
       jax.ShapeDtypeStruct((NPAD, 8), f32),
       jax.ShapeDtypeStruct((NPAD, 16), f32),
       jax.ShapeDtypeStruct((NPAD, 16), f32),
       jax.ShapeDtypeStruct((NPAD, H), f32)),
      (x0, sxa, sxb, dis, xn_raw, cw["w0x3"], cw["w1x3"], cw["bz"], cw["bh"],
       cw["wh8"], cw["bh8"], cw["e6r"], cw["q36"], cw["p1"], cw["p2"],
       cw["cmask"]),
  )


def _gates_body(x, sxa, sxb, h, sh, dis, w0x3, w1x3, w0h2, w1h2, w0hh, bz,
                br, bh, z_o, pxh_o, grh_o):
  xv = x[...]
  hv = h[...]
  d = dis[...]
  a = -(d * (sxa[...] + sxb[...]))
  u1 = (jnp.dot(xv, w0x3[...], preferred_element_type=f32)
        + jnp.dot(a, w1x3[...], preferred_element_type=f32))
  u2 = (jnp.dot(hv, w0h2[...], preferred_element_type=f32)
        + jnp.dot(-(d * sh[...]), w1h2[...], preferred_element_type=f32))
  z = jax.nn.sigmoid(u1[:, 0:H] + u2[:, 0:H] + bz[...])
  r = jax.nn.sigmoid(u1[:, H:2 * H] + u2[:, H:2 * H] + br[...])
  rh = r * hv
  pxh = (u1[:, 2 * H:3 * H]
         + jnp.dot(rh, w0hh[...], preferred_element_type=f32) + bh[...])
  real = (lax.broadcasted_iota(jnp.int32, (R, 1), 0)
          + pl.program_id(0) * R) < N
  z_o[...] = z
  pxh_o[...] = pxh
  grh_o[...] = jnp.where(real, d * rh, 0.0)


def _tc_gates(x, sxa, sxb, h, sh, dis, cw):
  return _tc_call(
      _gates_body,
      [_row_spec(16), _row_spec(16), _row_spec(16), _row_spec(H),
       _row_spec(H), _row_spec(1), _full_spec(16, 3 * H),
       _full_spec(16, 3 * H), _full_spec(H, 2 * H), _full_spec(H, 2 * H),
       _full_spec(H, H), _full_spec(1, H), _full_spec(1, H),
       _full_spec(1, H)],
      [_row_spec(H), _row_spec(H), _row_spec(H)],
      (jax.ShapeDtypeStruct((NPAD, H), f32),
       jax.ShapeDtypeStruct((NPAD, H), f32),
       jax.ShapeDtypeStruct((NPAD, H), f32)),
      (x, sxa, sxb, h, sh, dis, cw["w0x3"], cw["w1x3"], cw["w0h2"],
       cw["w1h2"], cw["w0hh"], cw["bz"], cw["br"], cw["bh"]),
  )


def _update_body(z, pxh, srh, h, dis, x, xn_raw, w1hh, wh8, bh8, e6r, q36,
                 p1, p2, cmask, h_o, u_o, x_o, gx_o, gh_o):
  d = dis[...]
  ht = jnp.tanh(pxh[...] + jnp.dot(-(d * srh[...]), w1hh[...],
                                   preferred_element_type=f32))
  zv = z[...]
  hn = zv * h[...] + (1.0 - zv) * ht
  u = jnp.dot(hn, wh8[...], preferred_element_type=f32) + bh8[...]
  c = {"e6r": e6r[...], "q36": q36[...], "p1": p1[...], "p2": p2[...],
       "cmask": cmask[...]}
  xn = _nextx(u, x[...], xn_raw[...], d, c)
  real = (lax.broadcasted_iota(jnp.int32, (R, 1), 0)
          + pl.program_id(0) * R) < N
  h_o[...] = hn
  u_o[...] = u
  x_o[...] = xn
  gx_o[...] = jnp.where(real, d * xn, 0.0)
  gh_o[...] = jnp.where(real, d * hn, 0.0)


def _tc_update(z, pxh, srh, h, dis, x, xn_raw, cw):
  return _tc_call(
      _update_body,
      [_row_spec(H), _row_spec(H), _row_spec(H), _row_spec(H), _row_spec(1),
       _row_spec(16), _row_spec(16), _full_spec(H, H), _full_spec(H, 8),
       _full_spec(1, 8), _full_spec(16, 8), _full_spec(16, 8),
       _full_spec(8, 16), _full_spec(8, 16), _full_spec(1, 16)],
      [_row_spec(H), _row_spec(8), _row_spec(16), _row_spec(16),
       _row_spec(H)],
      (jax.ShapeDtypeStruct((NPAD, H), f32),
       jax.ShapeDtypeStruct((NPAD, 8), f32),
       jax.ShapeDtypeStruct((NPAD, 16), f32),
       jax.ShapeDtypeStruct((NPAD, 16), f32),
       jax.ShapeDtypeStruct((NPAD, H), f32)),
      (z, pxh, srh, h, dis, x, xn_raw, cw["w1hh"], cw["wh8"], cw["bh8"],
       cw["e6r"], cw["q36"], cw["p1"], cw["p2"], cw["cmask"]),
  )


def _final_body(z, pxh, srh, h, dis, w1hh, wh8, bh8, u_o):
  d = dis[...]
  ht = jnp.tanh(pxh[...] + jnp.dot(-(d * srh[...]), w1hh[...],
                                   preferred_element_type=f32))
  zv = z[...]
  hn = zv * h[...] + (1.0 - zv) * ht
  u_o[...] = jnp.dot(hn, wh8[...], preferred_element_type=f32) + bh8[...]


def _tc_final(z, pxh, srh, h, dis, cw):
  return _tc_call(
      _final_body,
      [_row_spec(H), _row_spec(H), _row_spec(H), _row_spec(H), _row_spec(1),
       _full_spec(H, H), _full_spec(H, 8), _full_spec(1, 8)],
      [_row_spec(8)],
      (jax.ShapeDtypeStruct((NPAD, 8), f32),),
      (z, pxh, srh, h, dis, cw["w1hh"], cw["wh8"], cw["bh8"]),
  )


def kernel(X_seq, edge, W0_xz, W1_xz, b_xz, W0_hz, W1_hz, b_hz, W0_xr, W1_xr,
           b_xr, W0_hr, W1_hr, b_hr, W0_xh, W1_xh, b_xh, W0_hh, W1_hh, b_hh,
           W_head, b_head):
  xp = jnp.pad(X_seq, ((0, 0), (0, NPAD - N), (0, 16 - IN_F)))
  src0, dst0 = edge[0], edge[1]
  key = dst0 // QROWS
  ind = (key[None, :] == jnp.arange(4, dtype=jnp.int32)[:, None])
  posm = jnp.cumsum(ind.astype(jnp.int32), axis=1)
  pos = jnp.sum(posm * ind, axis=0) - 1
  fidx = jnp.where(pos < CAPB, key * CAPB + pos, EPB)
  bsrc = jnp.full((EPB,), N, jnp.int32).at[fidx].set(src0, mode="drop")
  bdstl = jnp.zeros((EPB,), jnp.int32).at[fidx].set(dst0 - key * QROWS,
                                                    mode="drop")
  bdstg = jnp.full((EPB,), N, jnp.int32).at[fidx].set(dst0, mode="drop")
  srcq = bsrc.reshape(4, 16, NBQ, BQ)
  dstq = bdstl.reshape(4, 16, NBQ, BQ)
  srcx = bsrc.reshape(32, NB_X, BATCH)
  dstx = bdstg.reshape(32, NB_X, BATCH)
  zeros16 = jnp.zeros((NPAD, 16), f32)
  ones_h = jnp.ones((BATCH, 16), f32)
  zeros_w = jnp.zeros((WQ, H), f32)

  def padw(w):
    return jnp.pad(w, ((0, 16 - IN_F), (0, 0)))

  cw = {
      "w0x3": padw(jnp.concatenate([W0_xz, W0_xr, W0_xh], axis=1)),
      "w1x3": padw(jnp.concatenate([W1_xz, W1_xr, W1_xh], axis=1)),
      "w0h2": jnp.concatenate([W0_hz, W0_hr], axis=1),
      "w1h2": jnp.concatenate([W1_hz, W1_hr], axis=1),
      "w0hh": W0_hh,
      "w1hh": W1_hh,
      "bz": (b_xz + b_hz)[None, :],
      "br": (b_xr + b_hr)[None, :],
      "bh": (b_xh + b_hh)[None, :],
      "wh8": jnp.pad(W_head, ((0, 0), (0, 8 - OUT_F))),
      "bh8": jnp.pad(b_head, (0, 8 - OUT_F))[None, :],
  }
  i16 = jnp.arange(16)
  i8 = jnp.arange(8)
  cw["p1"] = (i8[:, None] + 3 == i16[None, :]).astype(f32) * (i8 < 3)[:, None]
  cw["p2"] = (i8[:, None] + 8 == i16[None, :]).astype(f32) * (i8 < 3)[:, None]
  cw["q36"] = ((i16[:, None] == i8[None, :] + 3).astype(f32)
               * (i8 < 3)[None, :])
  cw["e6r"] = (i16[:, None] == 6).astype(f32) * jnp.ones((1, 8), f32)
  dyn = (i16 >= 3) & (i16 < 6) | (i16 >= 8) & (i16 < 11)
  cw["cmask"] = jnp.where(dyn, 0.0, 1.0)[None, :].astype(f32)

  dega, degb = _sc_deg(srcx, zeros16, ones_h)
  dis, gx = _tc_prep0(dega, degb, xp[0])
  sxa, sxb = _sc_seg16(gx, srcx, dstx, zeros16)
  h, u0, x, gx, gh = _tc_step0(xp[0], sxa, sxb, dis, xp[1], cw)
  us = [u0]
  for t in range(1, T):
    sxa, sxb = _sc_seg16(gx, srcx, dstx, zeros16)
    sh = _sc_seg128(gh, srcq, dstq, zeros_w)
    z, pxh, grh = _tc_gates(x, sxa, sxb, h, sh, dis, cw)
    srh = _sc_seg128(grh, srcq, dstq, zeros_w)
    if t < T - 1:
      h, u, x, gx, gh = _tc_update(z, pxh, srh, h, dis, x, xp[t + 1], cw)
    else:
      (u,) = _tc_final(z, pxh, srh, h, dis, cw)
    us.append(u)
  return jnp.stack([u[:N, :OUT_F] for u in us])
